# baseline (speedup 1.0000x reference)
import functools

import jax
import jax.numpy as jnp
from jax import lax
from jax.experimental import pallas as pl
from jax.experimental.pallas import tpu as pltpu
from jax.experimental.pallas import tpu_sc as plsc


def _sc_gather(table, ids, n_workers=32):
    m = ids.shape[0]
    d = table.shape[1]
    per_w = m // n_workers
    mesh = plsc.VectorSubcoreMesh(core_axis_name="c", subcore_axis_name="s")

    @functools.partial(
        pl.kernel,
        mesh=mesh,
        out_type=jax.ShapeDtypeStruct((m, d), table.dtype),
        scratch_types=[
            pltpu.VMEM((per_w,), jnp.int32),
            pltpu.VMEM((per_w, d), table.dtype),
            pltpu.SemaphoreType.DMA,
        ],
    )
    def gather_kernel(table_hbm, idx_hbm, out_hbm, idx_v, rows_v, sem):
        wid = lax.axis_index("s") * 2 + lax.axis_index("c")
        base = wid * per_w
        pltpu.sync_copy(idx_hbm.at[pl.ds(base, per_w)], idx_v)
        pltpu.async_copy(table_hbm.at[idx_v], rows_v, sem).wait()
        pltpu.sync_copy(rows_v, out_hbm.at[pl.ds(base, per_w)])

    return gather_kernel(table, ids)


def _fused_body(emb_ref, conc_ref, wc_ref, wr_ref, fused_ref, idx_ref):
    pre = emb_ref[...] + jnp.dot(
        conc_ref[...], wc_ref[...], preferred_element_type=jnp.float32
    )
    fused = jnp.tanh(pre)
    fused_ref[...] = fused.astype(jnp.bfloat16)
    logits = jnp.dot(fused, wr_ref[...], preferred_element_type=jnp.float32)
    e = logits.shape[1]
    iota = lax.broadcasted_iota(jnp.int32, logits.shape, 1)
    i1 = jnp.argmax(logits, axis=1).astype(jnp.int32)
    masked = jnp.where(iota == i1[:, None], -jnp.inf, logits)
    i2 = jnp.argmax(masked, axis=1).astype(jnp.int32)
    idx_ref[...] = jnp.concatenate([i1[:, None], i2[:, None]], axis=1)


def _fused_call(emb, conc, w_c, w_router, block_m=512, interpret=False):
    m, h = emb.shape
    e = w_router.shape[1]
    grid = (m // block_m,)
    return pl.pallas_call(
        _fused_body,
        grid=grid,
        in_specs=[
            pl.BlockSpec((block_m, h), lambda i: (i, 0)),
            pl.BlockSpec((block_m, h), lambda i: (i, 0)),
            pl.BlockSpec((h, h), lambda i: (0, 0)),
            pl.BlockSpec((h, e), lambda i: (0, 0)),
        ],
        out_specs=[
            pl.BlockSpec((block_m, h), lambda i: (i, 0)),
            pl.BlockSpec((block_m, 2), lambda i: (i, 0)),
        ],
        out_shape=[
            jax.ShapeDtypeStruct((m, h), jnp.bfloat16),
            jax.ShapeDtypeStruct((m, 2), jnp.int32),
        ],
        interpret=interpret,
    )(emb, conc, w_c, w_router)


def _proj_body(fused_ref, w_ref, b_ref, out_ref):
    m_idx = pl.program_id(1)
    block_m = out_ref.shape[0]
    a = fused_ref[pl.ds(m_idx * block_m, block_m), :]
    out_ref[...] = (
        jnp.dot(a, w_ref[...], preferred_element_type=jnp.float32) + b_ref[...]
    )


def _proj_call(fused_bf, w_out_bf, b_out_row, block_m=1024, block_n=1280,
               interpret=False):
    m, h = fused_bf.shape
    v = w_out_bf.shape[1]
    grid = (v // block_n, m // block_m)
    return pl.pallas_call(
        _proj_body,
        grid=grid,
        in_specs=[
            pl.BlockSpec((m, h), lambda n, i: (0, 0)),
            pl.BlockSpec((h, block_n), lambda n, i: (0, n)),
            pl.BlockSpec((1, block_n), lambda n, i: (0, n)),
        ],
        out_specs=pl.BlockSpec((block_m, block_n), lambda n, i: (i, n)),
        out_shape=jax.ShapeDtypeStruct((m, v), jnp.float32),
        interpret=interpret,
    )(fused_bf, w_out_bf, b_out_row)


def kernel(text_input, conceptual_input, token_table, W_c, W_router, W_out,
           b_out):
    b, s = text_input.shape
    m = b * s
    h = token_table.shape[1]
    v = W_out.shape[1]

    ids = text_input.reshape(m).astype(jnp.int32)
    conc = conceptual_input.reshape(m, h)

    emb = _sc_gather(token_table, ids)
    fused_bf, idx = _fused_call(emb, conc, W_c, W_router)
    logits = _proj_call(fused_bf, W_out.astype(jnp.bfloat16),
                        b_out.reshape(1, v))
    return logits.reshape(b, s, v), idx.reshape(b, s, 2)


# speedup vs baseline: 1.0067x; 1.0067x over previous
import functools

import jax
import jax.numpy as jnp
from jax import lax
from jax.experimental import pallas as pl
from jax.experimental.pallas import tpu as pltpu
from jax.experimental.pallas import tpu_sc as plsc


def _sc_gather(table, ids, n_workers=32, n_chunks=4):
    m = ids.shape[0]
    d = table.shape[1]
    per_w = m // n_workers
    rpc = per_w // n_chunks
    mesh = plsc.VectorSubcoreMesh(core_axis_name="c", subcore_axis_name="s")

    @functools.partial(
        pl.kernel,
        mesh=mesh,
        out_type=jax.ShapeDtypeStruct((m, d), table.dtype),
        scratch_types=[
            pltpu.VMEM((per_w,), jnp.int32),
            pltpu.VMEM((rpc, d), table.dtype),
            pltpu.VMEM((rpc, d), table.dtype),
            pltpu.SemaphoreType.DMA,
            pltpu.SemaphoreType.DMA,
            pltpu.SemaphoreType.DMA,
            pltpu.SemaphoreType.DMA,
        ],
    )
    def gather_kernel(table_hbm, idx_hbm, out_hbm, idx_v, rows0, rows1,
                      sg0, sg1, sw0, sw1):
        wid = lax.axis_index("s") * 2 + lax.axis_index("c")
        base = wid * per_w
        pltpu.sync_copy(idx_hbm.at[pl.ds(base, per_w)], idx_v)
        bufs = (rows0, rows1)
        gsems = (sg0, sg1)
        wsems = (sw0, sw1)
        gh = [None] * n_chunks
        wh = [None] * n_chunks
        for c in range(min(2, n_chunks)):
            gh[c] = pltpu.async_copy(
                table_hbm.at[idx_v.at[pl.ds(c * rpc, rpc)]], bufs[c % 2],
                gsems[c % 2])
        for c in range(n_chunks):
            b = c % 2
            gh[c].wait()
            wh[c] = pltpu.async_copy(
                bufs[b], out_hbm.at[pl.ds(base + c * rpc, rpc)], wsems[b])
            if c >= 1 and c + 1 < n_chunks:
                wh[c - 1].wait()
                gh[c + 1] = pltpu.async_copy(
                    table_hbm.at[idx_v.at[pl.ds((c + 1) * rpc, rpc)]],
                    bufs[(c + 1) % 2], gsems[(c + 1) % 2])
        for c in range(max(0, n_chunks - 2), n_chunks):
            wh[c].wait()

    return gather_kernel(table, ids)


def _fused_body(emb_ref, conc_ref, wc_ref, wr_ref, fused_ref, idx_ref):
    pre = emb_ref[...] + jnp.dot(
        conc_ref[...], wc_ref[...], preferred_element_type=jnp.float32
    )
    fused = jnp.tanh(pre)
    fused_ref[...] = fused.astype(jnp.bfloat16)
    logits = jnp.dot(fused, wr_ref[...], preferred_element_type=jnp.float32)
    e = logits.shape[1]
    iota = lax.broadcasted_iota(jnp.int32, logits.shape, 1)
    i1 = jnp.argmax(logits, axis=1).astype(jnp.int32)
    masked = jnp.where(iota == i1[:, None], -jnp.inf, logits)
    i2 = jnp.argmax(masked, axis=1).astype(jnp.int32)
    idx_ref[...] = jnp.concatenate([i1[:, None], i2[:, None]], axis=1)


def _fused_call(emb, conc, w_c, w_router, block_m=512, interpret=False):
    m, h = emb.shape
    e = w_router.shape[1]
    grid = (m // block_m,)
    return pl.pallas_call(
        _fused_body,
        grid=grid,
        in_specs=[
            pl.BlockSpec((block_m, h), lambda i: (i, 0)),
            pl.BlockSpec((block_m, h), lambda i: (i, 0)),
            pl.BlockSpec((h, h), lambda i: (0, 0)),
            pl.BlockSpec((h, e), lambda i: (0, 0)),
        ],
        out_specs=[
            pl.BlockSpec((block_m, h), lambda i: (i, 0)),
            pl.BlockSpec((block_m, 2), lambda i: (i, 0)),
        ],
        out_shape=[
            jax.ShapeDtypeStruct((m, h), jnp.bfloat16),
            jax.ShapeDtypeStruct((m, 2), jnp.int32),
        ],
        interpret=interpret,
    )(emb, conc, w_c, w_router)


def _proj_body(fused_ref, w_ref, b_ref, out_ref, wbf_ref):
    m_idx = pl.program_id(1)
    block_m = out_ref.shape[0]

    @pl.when(m_idx == 0)
    def _():
        wbf_ref[...] = w_ref[...].astype(jnp.bfloat16)

    a = fused_ref[pl.ds(m_idx * block_m, block_m), :]
    out_ref[...] = (
        jnp.dot(a, wbf_ref[...], preferred_element_type=jnp.float32)
        + b_ref[...]
    )


def _proj_call(fused_bf, w_out, b_out_row, block_m=1024, block_n=1280,
               interpret=False):
    m, h = fused_bf.shape
    v = w_out.shape[1]
    grid = (v // block_n, m // block_m)
    return pl.pallas_call(
        _proj_body,
        grid=grid,
        in_specs=[
            pl.BlockSpec((m, h), lambda n, i: (0, 0)),
            pl.BlockSpec((h, block_n), lambda n, i: (0, n)),
            pl.BlockSpec((1, block_n), lambda n, i: (0, n)),
        ],
        out_specs=pl.BlockSpec((block_m, block_n), lambda n, i: (i, n)),
        out_shape=jax.ShapeDtypeStruct((m, v), jnp.float32),
        scratch_shapes=[pltpu.VMEM((h, block_n), jnp.bfloat16)],
        interpret=interpret,
    )(fused_bf, w_out, b_out_row)


def kernel(text_input, conceptual_input, token_table, W_c, W_router, W_out,
           b_out):
    b, s = text_input.shape
    m = b * s
    h = token_table.shape[1]
    v = W_out.shape[1]

    ids = text_input.reshape(m).astype(jnp.int32)
    conc = conceptual_input.reshape(m, h)

    emb = _sc_gather(token_table, ids)
    fused_bf, idx = _fused_call(emb, conc, W_c, W_router)
    logits = _proj_call(fused_bf, W_out, b_out.reshape(1, v))
    return logits.reshape(b, s, v), idx.reshape(b, s, 2)


# speedup vs baseline: 1.0150x; 1.0083x over previous
import functools

import jax
import jax.numpy as jnp
from jax import lax
from jax.experimental import pallas as pl
from jax.experimental.pallas import tpu as pltpu
from jax.experimental.pallas import tpu_sc as plsc


def _sc_gather(table, ids, n_workers=32, n_chunks=4):
    m = ids.shape[0]
    d = table.shape[1]
    per_w = m // n_workers
    rpc = per_w // n_chunks
    mesh = plsc.VectorSubcoreMesh(core_axis_name="c", subcore_axis_name="s")

    @functools.partial(
        pl.kernel,
        mesh=mesh,
        out_type=jax.ShapeDtypeStruct((m, d), table.dtype),
        scratch_types=[
            pltpu.VMEM((per_w,), jnp.int32),
            pltpu.VMEM((rpc, d), table.dtype),
            pltpu.VMEM((rpc, d), table.dtype),
            pltpu.SemaphoreType.DMA,
            pltpu.SemaphoreType.DMA,
            pltpu.SemaphoreType.DMA,
            pltpu.SemaphoreType.DMA,
        ],
    )
    def gather_kernel(table_hbm, idx_hbm, out_hbm, idx_v, rows0, rows1,
                      sg0, sg1, sw0, sw1):
        wid = lax.axis_index("s") * 2 + lax.axis_index("c")
        base = wid * per_w
        pltpu.sync_copy(idx_hbm.at[pl.ds(base, per_w)], idx_v)
        bufs = (rows0, rows1)
        gsems = (sg0, sg1)
        wsems = (sw0, sw1)
        gh = [None] * n_chunks
        wh = [None] * n_chunks
        for c in range(min(2, n_chunks)):
            gh[c] = pltpu.async_copy(
                table_hbm.at[idx_v.at[pl.ds(c * rpc, rpc)]], bufs[c % 2],
                gsems[c % 2])
        for c in range(n_chunks):
            b = c % 2
            gh[c].wait()
            wh[c] = pltpu.async_copy(
                bufs[b], out_hbm.at[pl.ds(base + c * rpc, rpc)], wsems[b])
            if c >= 1 and c + 1 < n_chunks:
                wh[c - 1].wait()
                gh[c + 1] = pltpu.async_copy(
                    table_hbm.at[idx_v.at[pl.ds((c + 1) * rpc, rpc)]],
                    bufs[(c + 1) % 2], gsems[(c + 1) % 2])
        for c in range(max(0, n_chunks - 2), n_chunks):
            wh[c].wait()

    return gather_kernel(table, ids)


def _fused_body(emb_ref, conc_ref, wc_ref, wr_ref, fused_ref, idx_ref):
    pre = emb_ref[...] + jnp.dot(
        conc_ref[...], wc_ref[...], preferred_element_type=jnp.float32
    )
    fused = jnp.tanh(pre)
    fused_ref[...] = fused
    logits = jnp.dot(fused, wr_ref[...], preferred_element_type=jnp.float32)
    e = logits.shape[1]
    iota = lax.broadcasted_iota(jnp.int32, logits.shape, 1)
    i1 = jnp.argmax(logits, axis=1).astype(jnp.int32)
    masked = jnp.where(iota == i1[:, None], -jnp.inf, logits)
    i2 = jnp.argmax(masked, axis=1).astype(jnp.int32)
    idx_ref[...] = jnp.concatenate([i1[:, None], i2[:, None]], axis=1)


def _fused_call(emb, conc, w_c, w_router, block_m=512, interpret=False):
    m, h = emb.shape
    e = w_router.shape[1]
    grid = (m // block_m,)
    return pl.pallas_call(
        _fused_body,
        grid=grid,
        in_specs=[
            pl.BlockSpec((block_m, h), lambda i: (i, 0)),
            pl.BlockSpec((block_m, h), lambda i: (i, 0)),
            pl.BlockSpec((h, h), lambda i: (0, 0)),
            pl.BlockSpec((h, e), lambda i: (0, 0)),
        ],
        out_specs=[
            pl.BlockSpec((block_m, h), lambda i: (i, 0)),
            pl.BlockSpec((block_m, 2), lambda i: (i, 0)),
        ],
        out_shape=[
            jax.ShapeDtypeStruct((m, h), jnp.float32),
            jax.ShapeDtypeStruct((m, 2), jnp.int32),
        ],
        interpret=interpret,
    )(emb, conc, w_c, w_router)


def _proj_body(fused_ref, w_ref, b_ref, out_ref):
    m_idx = pl.program_id(1)
    block_m = out_ref.shape[0]
    a = fused_ref[pl.ds(m_idx * block_m, block_m), :]
    out_ref[...] = (
        jnp.dot(a, w_ref[...], preferred_element_type=jnp.float32)
        + b_ref[...]
    )


def _proj_call(fused_bf, w_out, b_out_row, block_m=1024, block_n=1280,
               interpret=False):
    m, h = fused_bf.shape
    v = w_out.shape[1]
    grid = (v // block_n, m // block_m)
    return pl.pallas_call(
        _proj_body,
        grid=grid,
        in_specs=[
            pl.BlockSpec((m, h), lambda n, i: (0, 0)),
            pl.BlockSpec((h, block_n), lambda n, i: (0, n)),
            pl.BlockSpec((1, block_n), lambda n, i: (0, n)),
        ],
        out_specs=pl.BlockSpec((block_m, block_n), lambda n, i: (i, n)),
        out_shape=jax.ShapeDtypeStruct((m, v), jnp.float32),
        interpret=interpret,
    )(fused_bf, w_out, b_out_row)


def kernel(text_input, conceptual_input, token_table, W_c, W_router, W_out,
           b_out):
    b, s = text_input.shape
    m = b * s
    h = token_table.shape[1]
    v = W_out.shape[1]

    ids = text_input.reshape(m).astype(jnp.int32)
    conc = conceptual_input.reshape(m, h)

    emb = _sc_gather(token_table, ids)
    fused_bf, idx = _fused_call(emb, conc, W_c, W_router)
    logits = _proj_call(fused_bf, W_out, b_out.reshape(1, v))
    return logits.reshape(b, s, v), idx.reshape(b, s, 2)
